# baseline (speedup 1.0000x reference)
import functools

import jax
import jax.numpy as jnp
from jax import lax
from jax.experimental import pallas as pl
from jax.experimental.pallas import tpu as pltpu
from jax.experimental.pallas import tpu_sc as plsc

VOCAB = 1000000
EMB = 16
BATCH = 16384
HIST = 200
B = BATCH * HIST

NC = 2
NS = 16
NW = NC * NS
CHUNK = 1024
PAIR = 2 * CHUNK
PPH = BATCH // PAIR
NPAIR = HIST * PPH
PPW = NPAIR // NW
NBODY = PPW // 2

_mesh = plsc.VectorSubcoreMesh(core_axis_name="c", subcore_axis_name="s")


@functools.partial(
    pl.kernel,
    mesh=_mesh,
    compiler_params=pltpu.CompilerParams(use_tc_tiling_on_sc=False,
                                         needs_layout_passes=False),
    out_type=jax.ShapeDtypeStruct((B * EMB,), jnp.float32),
    scratch_types=[
        pltpu.VMEM((2, CHUNK), jnp.int32),
        pltpu.VMEM((2, CHUNK), jnp.int32),
        pltpu.VMEM((CHUNK, EMB), jnp.float32),
        pltpu.VMEM((CHUNK, EMB), jnp.float32),
        pltpu.VMEM((PAIR * EMB,), jnp.float32),
        pltpu.VMEM((PAIR * EMB,), jnp.float32),
        pltpu.SemaphoreType.DMA,
        pltpu.SemaphoreType.DMA,
        pltpu.SemaphoreType.DMA,
        pltpu.SemaphoreType.DMA,
        pltpu.SemaphoreType.DMA,
        pltpu.SemaphoreType.DMA,
    ],
)
def _gather(idx_hbm, table_hbm, out_hbm,
            idx0, idx1, rowsa, rowsb, tbuf0, tbuf1,
            sga, sgb, sw0, sw1, si0, si1):
    wid = lax.axis_index("s") * NC + lax.axis_index("c")
    qbase = wid * PPW

    lane = lax.iota(jnp.int32, 16)
    cols = [(lane + d0) & 15 for d0 in range(16)]
    dsts = [((c >> 3) << 14) + ((c & 7) << 7) + lane for c in cols]

    def idx_row(q):
        return 2 * q

    def out_off(q):
        return (q // PPH) * (2 * 128 * 1024) + (q % PPH) * (16 * 1024)

    def transpose(rows, tbuf, half):
        def c0_body(blk, carry):
            c0 = blk * 16
            row0 = lane + c0
            ds = half * 8192 + ((c0 >> 7) << 10) + (c0 & 127)
            for d0 in range(16):
                v = plsc.load_gather(rows, [row0, cols[d0]])
                plsc.store_scatter(tbuf, [dsts[d0] + ds], v)
            return carry
        lax.fori_loop(0, CHUNK // 16, c0_body, 0)

    def fire_idx(q, idxp, si):
        pltpu.async_copy(idx_hbm.at[pl.ds(idx_row(q), 2), :], idxp, si)

    def drain_idx(q, idxp, si):
        pltpu.make_async_copy(idx_hbm.at[pl.ds(idx_row(q), 2), :],
                              idxp, si).wait()

    def fire_gather(idxp, half, rows, sg):
        pltpu.async_copy(table_hbm.at[idxp.at[half]], rows, sg)

    def wait_gather(idxp, half, rows, sg):
        pltpu.make_async_copy(table_hbm.at[idxp.at[half]], rows, sg).wait()

    def store(q, tbuf, sw):
        o = out_off(q)
        pltpu.async_copy(tbuf.at[pl.ds(0, 16384)],
                         out_hbm.at[pl.ds(o, 16384)], sw)
        pltpu.async_copy(tbuf.at[pl.ds(16384, 16384)],
                         out_hbm.at[pl.ds(o + 128 * 1024, 16384)], sw)

    def drain_store(q, tbuf, sw):
        o = out_off(q)
        pltpu.make_async_copy(tbuf.at[pl.ds(0, 16384)],
                              out_hbm.at[pl.ds(o, 16384)], sw).wait()
        pltpu.make_async_copy(tbuf.at[pl.ds(16384, 16384)],
                              out_hbm.at[pl.ds(o + 128 * 1024, 16384)], sw).wait()

    fire_idx(qbase + 0, idx0, si0)
    fire_idx(qbase + 1, idx1, si1)
    drain_idx(qbase + 0, idx0, si0)
    fire_gather(idx0, 0, rowsa, sga)
    fire_gather(idx0, 1, rowsb, sgb)

    def body(g, carry):
        q0 = qbase + 2 * g
        q1 = q0 + 1

        wait_gather(idx0, 0, rowsa, sga)

        @pl.when(g > 0)
        def _():
            drain_store(q0, tbuf0, sw0)

        transpose(rowsa, tbuf0, 0)
        drain_idx(q1, idx1, si1)
        fire_gather(idx1, 0, rowsa, sga)

        wait_gather(idx0, 1, rowsb, sgb)
        transpose(rowsb, tbuf0, 1)
        fire_gather(idx1, 1, rowsb, sgb)
        store(q0, tbuf0, sw0)

        @pl.when(g < NBODY - 1)
        def _():
            fire_idx(q0 + 2, idx0, si0)

        wait_gather(idx1, 0, rowsa, sga)

        @pl.when(g > 0)
        def _():
            drain_store(q1, tbuf1, sw1)

        transpose(rowsa, tbuf1, 0)

        @pl.when(g < NBODY - 1)
        def _():
            drain_idx(q0 + 2, idx0, si0)
            fire_gather(idx0, 0, rowsa, sga)

        wait_gather(idx1, 1, rowsb, sgb)
        transpose(rowsb, tbuf1, 1)

        @pl.when(g < NBODY - 1)
        def _():
            fire_gather(idx0, 1, rowsb, sgb)
            fire_idx(q1 + 2, idx1, si1)

        store(q1, tbuf1, sw1)
        return carry

    lax.fori_loop(0, NBODY, body, 0)
    drain_store(qbase, tbuf0, sw0)
    drain_store(qbase + 1, tbuf1, sw1)


def kernel(x, table):
    flat_idx = x.T.reshape(B // CHUNK, CHUNK)
    out = _gather(flat_idx, table)
    out5 = out.reshape(HIST, 2, 128, 8, 128)
    return out5.transpose((2, 4, 0, 1, 3)).reshape(BATCH, HIST, EMB)


# speedup vs baseline: 1.3480x; 1.3480x over previous
import functools

import jax
import jax.numpy as jnp
from jax import lax
from jax.experimental import pallas as pl
from jax.experimental.pallas import tpu as pltpu
from jax.experimental.pallas import tpu_sc as plsc

VOCAB = 1000000
EMB = 16
BATCH = 16384
HIST = 200
B = BATCH * HIST

NC = 2
NS = 16
NW = NC * NS
CHUNK = 1024
PAIR = 2 * CHUNK
PPH = BATCH // PAIR
NPAIR = HIST * PPH
PPW = NPAIR // NW
NBODY = PPW // 2

_mesh = plsc.VectorSubcoreMesh(core_axis_name="c", subcore_axis_name="s")


@functools.partial(
    pl.kernel,
    mesh=_mesh,
    compiler_params=pltpu.CompilerParams(use_tc_tiling_on_sc=False,
                                         needs_layout_passes=False),
    out_type=jax.ShapeDtypeStruct((B * EMB,), jnp.float32),
    scratch_types=[
        pltpu.VMEM((2, CHUNK), jnp.int32),
        pltpu.VMEM((2, CHUNK), jnp.int32),
        pltpu.VMEM((CHUNK, EMB), jnp.float32),
        pltpu.VMEM((CHUNK, EMB), jnp.float32),
        pltpu.VMEM((PAIR * EMB,), jnp.float32),
        pltpu.VMEM((PAIR * EMB,), jnp.float32),
        pltpu.SemaphoreType.DMA,
        pltpu.SemaphoreType.DMA,
        pltpu.SemaphoreType.DMA,
        pltpu.SemaphoreType.DMA,
        pltpu.SemaphoreType.DMA,
        pltpu.SemaphoreType.DMA,
    ],
)
def _gather(idx_hbm, table_hbm, out_hbm,
            idx0, idx1, rowsa, rowsb, tbuf0, tbuf1,
            sga, sgb, sw0, sw1, si0, si1):
    wid = lax.axis_index("s") * NC + lax.axis_index("c")
    qbase = wid * PPW

    lane = lax.iota(jnp.int32, 16)
    cols = [(lane + d0) & 15 for d0 in range(16)]
    dsts = [((c >> 3) << 14) + ((c & 7) << 7) + lane for c in cols]

    def idx_row(q):
        return 2 * q

    def out_off(q):
        return (q // PPH) * (2 * 128 * 1024) + (q % PPH) * (16 * 1024)

    def transpose(rows, tbuf, half):
        def c0_body(blk, carry):
            c0 = blk * 16
            row0 = lane + c0
            ds = half * 8192 + ((c0 >> 7) << 10) + (c0 & 127)
            vs = [plsc.load_gather(rows, [row0, cols[d0]])
                  for d0 in range(16)]
            for d0 in range(16):
                plsc.store_scatter(tbuf, [dsts[d0] + ds], vs[d0])
            return carry
        lax.fori_loop(0, CHUNK // 16, c0_body, 0)

    def fire_idx(q, idxp, si):
        pltpu.async_copy(idx_hbm.at[pl.ds(idx_row(q), 2), :], idxp, si)

    def drain_idx(q, idxp, si):
        pltpu.make_async_copy(idx_hbm.at[pl.ds(idx_row(q), 2), :],
                              idxp, si).wait()

    def fire_gather(idxp, half, rows, sg):
        pltpu.async_copy(table_hbm.at[idxp.at[half]], rows, sg)

    def wait_gather(idxp, half, rows, sg):
        pltpu.make_async_copy(table_hbm.at[idxp.at[half]], rows, sg).wait()

    def store(q, tbuf, sw):
        o = out_off(q)
        pltpu.async_copy(tbuf.at[pl.ds(0, 16384)],
                         out_hbm.at[pl.ds(o, 16384)], sw)
        pltpu.async_copy(tbuf.at[pl.ds(16384, 16384)],
                         out_hbm.at[pl.ds(o + 128 * 1024, 16384)], sw)

    def drain_store(q, tbuf, sw):
        o = out_off(q)
        pltpu.make_async_copy(tbuf.at[pl.ds(0, 16384)],
                              out_hbm.at[pl.ds(o, 16384)], sw).wait()
        pltpu.make_async_copy(tbuf.at[pl.ds(16384, 16384)],
                              out_hbm.at[pl.ds(o + 128 * 1024, 16384)], sw).wait()

    fire_idx(qbase + 0, idx0, si0)
    fire_idx(qbase + 1, idx1, si1)
    drain_idx(qbase + 0, idx0, si0)
    fire_gather(idx0, 0, rowsa, sga)
    fire_gather(idx0, 1, rowsb, sgb)

    def body(g, carry):
        q0 = qbase + 2 * g
        q1 = q0 + 1

        wait_gather(idx0, 0, rowsa, sga)

        @pl.when(g > 0)
        def _():
            drain_store(q0, tbuf0, sw0)

        transpose(rowsa, tbuf0, 0)
        drain_idx(q1, idx1, si1)
        fire_gather(idx1, 0, rowsa, sga)

        wait_gather(idx0, 1, rowsb, sgb)
        transpose(rowsb, tbuf0, 1)
        fire_gather(idx1, 1, rowsb, sgb)
        store(q0, tbuf0, sw0)

        @pl.when(g < NBODY - 1)
        def _():
            fire_idx(q0 + 2, idx0, si0)

        wait_gather(idx1, 0, rowsa, sga)

        @pl.when(g > 0)
        def _():
            drain_store(q1, tbuf1, sw1)

        transpose(rowsa, tbuf1, 0)

        @pl.when(g < NBODY - 1)
        def _():
            drain_idx(q0 + 2, idx0, si0)
            fire_gather(idx0, 0, rowsa, sga)

        wait_gather(idx1, 1, rowsb, sgb)
        transpose(rowsb, tbuf1, 1)

        @pl.when(g < NBODY - 1)
        def _():
            fire_gather(idx0, 1, rowsb, sgb)
            fire_idx(q1 + 2, idx1, si1)

        store(q1, tbuf1, sw1)
        return carry

    lax.fori_loop(0, NBODY, body, 0)
    drain_store(qbase, tbuf0, sw0)
    drain_store(qbase + 1, tbuf1, sw1)


def kernel(x, table):
    flat_idx = x.T.reshape(B // CHUNK, CHUNK)
    out = _gather(flat_idx, table)
    out5 = out.reshape(HIST, 2, 128, 8, 128)
    return out5.transpose((2, 4, 0, 1, 3)).reshape(BATCH, HIST, EMB)
